# baseline (speedup 1.0000x reference)
import functools

import jax
import jax.numpy as jnp
import numpy as np
from jax import lax
from jax.experimental import pallas as pl
from jax.experimental.pallas import tpu as pltpu
from jax.experimental.pallas import tpu_sc as plsc

N = 10000
E = 160000
DIN = 256
H = 1024
SEQ = 256
NCLS = 128
B = 64

NCORES = 2
NSUB = 16
NW = NCORES * NSUB

LN = 128
EB = 128
NB_E = 40
NB_P = 4
NPAD = 10112
RPT = NPAD // NSUB
NPOOL = 128
BN_ROWS = 400
RB = N // BN_ROWS

_mesh = functools.partial(
    plsc.VectorSubcoreMesh,
    core_axis_name="c", subcore_axis_name="s",
    num_cores=NCORES, num_subcores=NSUB)


def _zero_slice(zb, acc, r0, rows):
    done = 0
    while done < rows:
        step = min(40, rows - done)
        pltpu.sync_copy(zb.at[pl.ds(0, step)], acc.at[pl.ds(r0 + done, step)])
        done += step


def _sc_poolagg(xin, src3, dst3, z128, tok, n_chunks, nb, npad_out,
                rpt_out):
    assert nb % 2 == 0
    niter = nb // 2

    def body(x_h, src_h, dst_h, z_h, tok_h, outp,
             si2, di2, g0, zb, tokv, acc, s0, s1):
        scid = lax.axis_index("c")
        sid = lax.axis_index("s")
        wid = scid * NSUB + sid
        pltpu.sync_copy(z_h.at[pl.ds(0, 40)], zb)
        pltpu.sync_copy(tok_h, tokv)
        r0 = sid * rpt_out
        for c in range(n_chunks):
            _zero_slice(zb, acc, r0, rpt_out)
            plsc.subcore_barrier()

            def step(i, carry):
                pltpu.sync_copy(
                    src_h.at[wid].at[pl.ds(i * 2 * EB, 2 * EB)], si2)
                pltpu.sync_copy(
                    dst_h.at[wid].at[pl.ds(i * 2 * EB, 2 * EB)], di2)
                pltpu.async_copy(x_h.at[c].at[si2], g0, s0).wait()
                pltpu.sync_copy(g0, acc.at[di2], add=True)
                return carry
            lax.fori_loop(0, niter, step, 0)
            plsc.subcore_barrier()
            pltpu.sync_copy(acc.at[pl.ds(r0, rpt_out)],
                            outp.at[scid].at[c].at[pl.ds(r0, rpt_out)])

    fn = pl.kernel(
        body,
        out_type=jax.ShapeDtypeStruct(
            (NCORES, n_chunks, npad_out, LN), jnp.float32),
        mesh=_mesh(),
        scratch_types=[
            pltpu.VMEM((2 * EB,), jnp.int32),
            pltpu.VMEM((2 * EB,), jnp.int32),
            pltpu.VMEM((2 * EB, LN), jnp.float32),
            pltpu.VMEM((40, LN), jnp.float32),
            pltpu.VMEM((8, LN), jnp.float32),
            pltpu.VMEM_SHARED((npad_out, LN), jnp.float32),
            pltpu.SemaphoreType.DMA,
            pltpu.SemaphoreType.DMA,
        ],
    )
    return fn(xin, src3.reshape(NW, nb * EB),
              dst3.reshape(NW, nb * EB), z128, tok)


def _tc_prep(x_res, degp):
    def body(xr, dp, x1p, dinv_o):
        c = pl.program_id(1)
        deg = dp[0, 0, :, 0] + dp[1, 0, :, 0] + 1.0
        dv = lax.rsqrt(jnp.maximum(deg, 1.0))[:, None]
        dinv_o[...] = dv
        cols = lax.broadcasted_iota(jnp.int32, (BN_ROWS, LN), 1)
        dcol = jnp.where(cols == 0, dv, 0.0)
        x1p[0] = jnp.where(c == 2, dcol, xr[...] * dv)

    return pl.pallas_call(
        body,
        grid=(RB, 3),
        in_specs=[
            pl.BlockSpec((BN_ROWS, LN), lambda i, c: (i, jnp.minimum(c, 1))),
            pl.BlockSpec((NCORES, 1, BN_ROWS, LN), lambda i, c: (0, 0, i, 0)),
        ],
        out_specs=[
            pl.BlockSpec((1, BN_ROWS, LN), lambda i, c: (c, i, 0)),
            pl.BlockSpec((BN_ROWS, 1), lambda i, c: (i, 0)),
        ],
        out_shape=[
            jax.ShapeDtypeStruct((3, N, LN), jnp.float32),
            jax.ShapeDtypeStruct((N, 1), jnp.float32),
        ],
    )(x_res, degp)


def _tc_matmul(aggp, xp, dinv, rowsum, wc, bvec, n_chunks):
    def body(ap, xb, dv, rs, w, bv, h_ref, stats_ref, rs_out, stats_acc):
        i = pl.program_id(0)
        c = pl.program_id(1)
        aggc = (ap[0, 0] + ap[1, 0] + xb[0]) * dv[...]
        part = jnp.dot(aggc, w[0], preferred_element_type=jnp.float32)

        @pl.when(c == 0)
        def _():
            h_ref[...] = rs[...] * bv[...] + part

        @pl.when(c != 0)
        def _():
            h_ref[...] += part

        @pl.when(c == n_chunks - 1)
        def _():
            hb = h_ref[...]
            st = jnp.stack([jnp.sum(hb, 0), jnp.sum(hb * hb, 0)])

            @pl.when(i == 0)
            def _():
                stats_acc[...] = st

            @pl.when(i != 0)
            def _():
                stats_acc[...] += st
            rs_out[...] = aggc[:, 0:1]

        stats_ref[...] = stats_acc[...]

    return pl.pallas_call(
        body,
        grid=(RB, n_chunks),
        in_specs=[
            pl.BlockSpec((NCORES, 1, BN_ROWS, LN), lambda i, c: (0, c, i, 0)),
            pl.BlockSpec((1, BN_ROWS, LN), lambda i, c: (c, i, 0)),
            pl.BlockSpec((BN_ROWS, 1), lambda i, c: (i, 0)),
            pl.BlockSpec((BN_ROWS, 1), lambda i, c: (i, 0)),
            pl.BlockSpec((1, LN, H), lambda i, c: (c, 0, 0)),
            pl.BlockSpec((1, H), lambda i, c: (0, 0)),
        ],
        out_specs=[
            pl.BlockSpec((BN_ROWS, H), lambda i, c: (i, 0)),
            pl.BlockSpec((2, H), lambda i, c: (0, 0)),
            pl.BlockSpec((BN_ROWS, 1), lambda i, c: (i, 0)),
        ],
        out_shape=[
            jax.ShapeDtypeStruct((N, H), jnp.float32),
            jax.ShapeDtypeStruct((2, H), jnp.float32),
            jax.ShapeDtypeStruct((N, 1), jnp.float32),
        ],
        scratch_shapes=[pltpu.VMEM((2, H), jnp.float32)],
    )(aggp, xp, dinv, rowsum, wc, bvec)


def _tc_finalize(h, stats, g, be, dinv, scale):
    def body(hb, st, gb, bb, dv, out):
        mu = st[0:1, :] / N
        var = st[1:2, :] / N - mu * mu
        y = (hb[...] - mu) * lax.rsqrt(var + 1e-5) * gb[...] + bb[...]
        y = jnp.maximum(y, 0.0)
        if scale:
            y = y * dv[...]
        out[0] = y

    return pl.pallas_call(
        body,
        grid=(RB, H // LN),
        in_specs=[
            pl.BlockSpec((BN_ROWS, LN), lambda i, c: (i, c)),
            pl.BlockSpec((2, LN), lambda i, c: (0, c)),
            pl.BlockSpec((1, LN), lambda i, c: (0, c)),
            pl.BlockSpec((1, LN), lambda i, c: (0, c)),
            pl.BlockSpec((BN_ROWS, 1), lambda i, c: (i, 0)),
        ],
        out_specs=pl.BlockSpec((1, BN_ROWS, LN), lambda i, c: (c, i, 0)),
        out_shape=jax.ShapeDtypeStruct((H // LN, N, LN), jnp.float32),
    )(h, stats, g, be, dinv)


def _tc_head(poolp, xbat2, x_emb, wf, bf, gf, bef, wl, bl):
    def body(pp, cp, xe, wfb, bfb, gfb, befb, wlb, blb, out):
        psum = pp[0] + pp[1]
        bids = lax.broadcasted_iota(jnp.int32, (B, 80, LN), 0)
        cnt = jnp.sum((cp[...][None, :, :] == bids).astype(jnp.float32),
                      axis=(1, 2))
        cntc = jnp.maximum(cnt, 1.0)[:, None]
        y = jnp.dot(xe[...], wfb[...], preferred_element_type=jnp.float32)
        y = y + bfb[...]
        mu = jnp.mean(y, axis=0, keepdims=True)
        var = jnp.mean(y * y, axis=0, keepdims=True) - mu * mu
        ybn = (y - mu) * lax.rsqrt(var + 1e-5) * gfb[...] + befb[...]
        zz = jnp.broadcast_to(blb[...], (B, NCLS))
        for c in range(H // LN):
            zc = psum[c, :B, :] / cntc + ybn[:, c * LN:(c + 1) * LN]
            zz = zz + jnp.dot(zc, wlb[c], preferred_element_type=jnp.float32)
        s = jax.nn.sigmoid(zz)
        out[:, 0:NCLS] = s
        out[:, NCLS:2 * NCLS] = s

    z4 = lambda i: (0, 0, 0, 0)
    z3 = lambda i: (0, 0, 0)
    z2 = lambda i: (0, 0)
    return pl.pallas_call(
        body,
        grid=(1,),
        in_specs=[
            pl.BlockSpec((NCORES, H // LN, NPOOL, LN), z4),
            pl.BlockSpec((80, LN), z2),
            pl.BlockSpec((B, SEQ), z2),
            pl.BlockSpec((SEQ, H), z2),
            pl.BlockSpec((1, H), z2),
            pl.BlockSpec((1, H), z2),
            pl.BlockSpec((1, H), z2),
            pl.BlockSpec((H // LN, LN, NCLS), z3),
            pl.BlockSpec((1, NCLS), z2),
        ],
        out_specs=pl.BlockSpec((B, 2 * NCLS), z2),
        out_shape=jax.ShapeDtypeStruct((B, 2 * NCLS), jnp.float32),
    )(poolp, xbat2, x_emb, wf, bf, gf, bef, wl, bl)


def kernel(x_res, x_emb_seq, edge_index, edge_attr, x_batch,
           W1, b1, g1, be1, W2, b2, g2, be2, W3, b3, g3, be3,
           Wf, bf, gf, bef, Wl, bl):
    del edge_attr
    f32 = jnp.float32
    i32 = jnp.int32

    src = edge_index[0]
    dst = edge_index[1]
    epad = NW * NB_E * EB - E
    src3 = jnp.concatenate([src, jnp.zeros((epad,), i32)]).reshape(
        NW, NB_E, EB)
    dst3 = jnp.concatenate([dst, jnp.full((epad,), N, i32)]).reshape(
        NW, NB_E, EB)
    npad_n = NW * NB_P * EB - N
    psrc3 = jnp.concatenate(
        [jnp.arange(N, dtype=i32), jnp.zeros((npad_n,), i32)]).reshape(
            NW, NB_P, EB)
    pdst3 = jnp.concatenate(
        [x_batch, jnp.full((npad_n,), B, i32)]).reshape(NW, NB_P, EB)

    z128 = jnp.asarray(np.zeros((128, LN), np.float32))
    _onc = np.zeros((N, LN), np.float32)
    _onc[:, 0] = 1.0
    ones_nc = jnp.asarray(_onc)

    w1aug = jnp.concatenate(
        [W1.reshape(2, LN, H),
         jnp.concatenate([b1[None, :], jnp.zeros((LN - 1, H), f32)],
                         axis=0)[None]], axis=0)
    w2c = W2.reshape(H // LN, LN, H)
    w3c = W3.reshape(H // LN, LN, H)
    zcol = jnp.zeros((N, 1), f32)

    tok = z128[:8]
    degp = _sc_poolagg(ones_nc[None], src3, dst3, z128, tok, 1,
                       NB_E, NPAD, RPT)
    tok = degp[0, 0, :8]
    x1p, dinv = _tc_prep(x_res, degp)

    agg1p = _sc_poolagg(x1p, src3, dst3, z128, tok, 3, NB_E, NPAD, RPT)
    tok = agg1p[0, 0, :8]
    h1, st1, rowsum = _tc_matmul(agg1p, x1p, dinv, zcol, w1aug,
                                 jnp.zeros((1, H), f32), 3)
    x2p = _tc_finalize(h1, st1, g1.reshape(1, H), be1.reshape(1, H),
                       dinv, True)

    agg2p = _sc_poolagg(x2p, src3, dst3, z128, tok, H // LN, NB_E, NPAD,
                        RPT)
    tok = agg2p[0, 0, :8]
    h2, st2, _ = _tc_matmul(agg2p, x2p, dinv, rowsum, w2c,
                            b2.reshape(1, H), H // LN)
    x3p = _tc_finalize(h2, st2, g2.reshape(1, H), be2.reshape(1, H),
                       dinv, True)

    agg3p = _sc_poolagg(x3p, src3, dst3, z128, tok, H // LN, NB_E, NPAD,
                        RPT)
    tok = agg3p[0, 0, :8]
    h3, st3, _ = _tc_matmul(agg3p, x3p, dinv, rowsum, w3c,
                            b3.reshape(1, H), H // LN)
    x4p = _tc_finalize(h3, st3, g3.reshape(1, H), be3.reshape(1, H),
                       dinv, False)

    poolp = _sc_poolagg(x4p, psrc3, pdst3, z128, tok, H // LN,
                        NB_P, NPOOL, NPOOL // NSUB)
    xbat2 = jnp.concatenate(
        [x_batch, jnp.full((80 * LN - N,), B, i32)]).reshape(80, LN)
    return _tc_head(poolp, xbat2, x_emb_seq,
                    Wf, bf.reshape(1, H), gf.reshape(1, H),
                    bef.reshape(1, H), Wl.reshape(H // LN, LN, NCLS),
                    bl.reshape(1, NCLS))


# speedup vs baseline: 1.0369x; 1.0369x over previous
import functools

import jax
import jax.numpy as jnp
import numpy as np
from jax import lax
from jax.experimental import pallas as pl
from jax.experimental.pallas import tpu as pltpu
from jax.experimental.pallas import tpu_sc as plsc

N = 10000
E = 160000
DIN = 256
H = 1024
SEQ = 256
NCLS = 128
B = 64

NCORES = 2
NSUB = 16
NW = NCORES * NSUB

LN = 128
EB = 128
NB_E = 40
NB_P = 4
NPAD = 10112
RPT = NPAD // NSUB
NPOOL = 128
BN_ROWS = 400
RB = N // BN_ROWS

_mesh = functools.partial(
    plsc.VectorSubcoreMesh,
    core_axis_name="c", subcore_axis_name="s",
    num_cores=NCORES, num_subcores=NSUB)


def _zero_slice(zb, acc, r0, rows):
    done = 0
    while done < rows:
        step = min(40, rows - done)
        pltpu.sync_copy(zb.at[pl.ds(0, step)], acc.at[pl.ds(r0 + done, step)])
        done += step


def _sc_poolagg(xin, src3, dst3, z128, tok, n_chunks, nb, npad_out,
                rpt_out):
    assert nb % 2 == 0
    niter = nb // 2

    assert niter % 2 == 0
    niter2 = niter // 2

    def body(x_h, src_h, dst_h, z_h, tok_h, outp,
             sia, dia, sib, dib, g0, zb, tokv, acc, s0, sxa, sxb):
        scid = lax.axis_index("c")
        sid = lax.axis_index("s")
        wid = scid * NSUB + sid
        pltpu.sync_copy(z_h.at[pl.ds(0, 40)], zb)
        pltpu.sync_copy(tok_h, tokv)
        r0 = sid * rpt_out
        for c in range(n_chunks):
            _zero_slice(zb, acc, r0, rpt_out)
            plsc.subcore_barrier()

            def fire_idx(i, si, di, sx):
                pltpu.async_copy(
                    src_h.at[wid].at[pl.ds(i * 2 * EB, 2 * EB)], si, sx)
                pltpu.async_copy(
                    dst_h.at[wid].at[pl.ds(i * 2 * EB, 2 * EB)], di, sx)

            def wait_idx(i, si, di, sx):
                pltpu.make_async_copy(
                    src_h.at[wid].at[pl.ds(i * 2 * EB, 2 * EB)], si,
                    sx).wait()
                pltpu.make_async_copy(
                    dst_h.at[wid].at[pl.ds(i * 2 * EB, 2 * EB)], di,
                    sx).wait()

            fire_idx(0, sia, dia, sxa)

            def step(j, carry):
                i0 = j * 2
                i1 = i0 + 1
                wait_idx(i0, sia, dia, sxa)
                fire_idx(i1, sib, dib, sxb)
                pltpu.async_copy(x_h.at[c].at[sia], g0, s0).wait()
                pltpu.sync_copy(g0, acc.at[dia], add=True)
                wait_idx(i1, sib, dib, sxb)

                @pl.when(j < niter2 - 1)
                def _():
                    fire_idx(i0 + 2, sia, dia, sxa)
                pltpu.async_copy(x_h.at[c].at[sib], g0, s0).wait()
                pltpu.sync_copy(g0, acc.at[dib], add=True)
                return carry
            lax.fori_loop(0, niter2, step, 0)
            plsc.subcore_barrier()
            pltpu.sync_copy(acc.at[pl.ds(r0, rpt_out)],
                            outp.at[scid].at[c].at[pl.ds(r0, rpt_out)])

    fn = pl.kernel(
        body,
        out_type=jax.ShapeDtypeStruct(
            (NCORES, n_chunks, npad_out, LN), jnp.float32),
        mesh=_mesh(),
        scratch_types=[
            pltpu.VMEM((2 * EB,), jnp.int32),
            pltpu.VMEM((2 * EB,), jnp.int32),
            pltpu.VMEM((2 * EB,), jnp.int32),
            pltpu.VMEM((2 * EB,), jnp.int32),
            pltpu.VMEM((2 * EB, LN), jnp.float32),
            pltpu.VMEM((40, LN), jnp.float32),
            pltpu.VMEM((8, LN), jnp.float32),
            pltpu.VMEM_SHARED((npad_out, LN), jnp.float32),
            pltpu.SemaphoreType.DMA,
            pltpu.SemaphoreType.DMA,
            pltpu.SemaphoreType.DMA,
        ],
    )
    return fn(xin, src3.reshape(NW, nb * EB),
              dst3.reshape(NW, nb * EB), z128, tok)


def _tc_prep(x_res, degp):
    def body(xr, dp, x1p, dinv_o):
        c = pl.program_id(1)
        deg = dp[0, 0, :, 0] + dp[1, 0, :, 0] + 1.0
        dv = lax.rsqrt(jnp.maximum(deg, 1.0))[:, None]
        dinv_o[...] = dv
        cols = lax.broadcasted_iota(jnp.int32, (BN_ROWS, LN), 1)
        dcol = jnp.where(cols == 0, dv, 0.0)
        x1p[0] = jnp.where(c == 2, dcol, xr[...] * dv)

    return pl.pallas_call(
        body,
        grid=(RB, 3),
        in_specs=[
            pl.BlockSpec((BN_ROWS, LN), lambda i, c: (i, jnp.minimum(c, 1))),
            pl.BlockSpec((NCORES, 1, BN_ROWS, LN), lambda i, c: (0, 0, i, 0)),
        ],
        out_specs=[
            pl.BlockSpec((1, BN_ROWS, LN), lambda i, c: (c, i, 0)),
            pl.BlockSpec((BN_ROWS, 1), lambda i, c: (i, 0)),
        ],
        out_shape=[
            jax.ShapeDtypeStruct((3, N, LN), jnp.float32),
            jax.ShapeDtypeStruct((N, 1), jnp.float32),
        ],
    )(x_res, degp)


def _tc_matmul(aggp, xp, dinv, rowsum, wc, bvec, n_chunks):
    def body(ap, xb, dv, rs, w, bv, h_ref, stats_ref, rs_out, stats_acc):
        i = pl.program_id(0)
        c = pl.program_id(1)
        aggc = (ap[0, 0] + ap[1, 0] + xb[0]) * dv[...]
        part = jnp.dot(aggc, w[0], preferred_element_type=jnp.float32)

        @pl.when(c == 0)
        def _():
            h_ref[...] = rs[...] * bv[...] + part

        @pl.when(c != 0)
        def _():
            h_ref[...] += part

        @pl.when(c == n_chunks - 1)
        def _():
            hb = h_ref[...]
            st = jnp.stack([jnp.sum(hb, 0), jnp.sum(hb * hb, 0)])

            @pl.when(i == 0)
            def _():
                stats_acc[...] = st

            @pl.when(i != 0)
            def _():
                stats_acc[...] += st
            rs_out[...] = aggc[:, 0:1]

        stats_ref[...] = stats_acc[...]

    return pl.pallas_call(
        body,
        grid=(RB, n_chunks),
        in_specs=[
            pl.BlockSpec((NCORES, 1, BN_ROWS, LN), lambda i, c: (0, c, i, 0)),
            pl.BlockSpec((1, BN_ROWS, LN), lambda i, c: (c, i, 0)),
            pl.BlockSpec((BN_ROWS, 1), lambda i, c: (i, 0)),
            pl.BlockSpec((BN_ROWS, 1), lambda i, c: (i, 0)),
            pl.BlockSpec((1, LN, H), lambda i, c: (c, 0, 0)),
            pl.BlockSpec((1, H), lambda i, c: (0, 0)),
        ],
        out_specs=[
            pl.BlockSpec((BN_ROWS, H), lambda i, c: (i, 0)),
            pl.BlockSpec((2, H), lambda i, c: (0, 0)),
            pl.BlockSpec((BN_ROWS, 1), lambda i, c: (i, 0)),
        ],
        out_shape=[
            jax.ShapeDtypeStruct((N, H), jnp.float32),
            jax.ShapeDtypeStruct((2, H), jnp.float32),
            jax.ShapeDtypeStruct((N, 1), jnp.float32),
        ],
        scratch_shapes=[pltpu.VMEM((2, H), jnp.float32)],
    )(aggp, xp, dinv, rowsum, wc, bvec)


def _tc_finalize(h, stats, g, be, dinv, scale):
    def body(hb, st, gb, bb, dv, out):
        mu = st[0:1, :] / N
        var = st[1:2, :] / N - mu * mu
        y = (hb[...] - mu) * lax.rsqrt(var + 1e-5) * gb[...] + bb[...]
        y = jnp.maximum(y, 0.0)
        if scale:
            y = y * dv[...]
        out[0] = y

    return pl.pallas_call(
        body,
        grid=(RB, H // LN),
        in_specs=[
            pl.BlockSpec((BN_ROWS, LN), lambda i, c: (i, c)),
            pl.BlockSpec((2, LN), lambda i, c: (0, c)),
            pl.BlockSpec((1, LN), lambda i, c: (0, c)),
            pl.BlockSpec((1, LN), lambda i, c: (0, c)),
            pl.BlockSpec((BN_ROWS, 1), lambda i, c: (i, 0)),
        ],
        out_specs=pl.BlockSpec((1, BN_ROWS, LN), lambda i, c: (c, i, 0)),
        out_shape=jax.ShapeDtypeStruct((H // LN, N, LN), jnp.float32),
    )(h, stats, g, be, dinv)


def _tc_head(poolp, xbat2, x_emb, wf, bf, gf, bef, wl, bl):
    def body(pp, cp, xe, wfb, bfb, gfb, befb, wlb, blb, out):
        psum = pp[0] + pp[1]
        bids = lax.broadcasted_iota(jnp.int32, (B, 80, LN), 0)
        cnt = jnp.sum((cp[...][None, :, :] == bids).astype(jnp.float32),
                      axis=(1, 2))
        cntc = jnp.maximum(cnt, 1.0)[:, None]
        y = jnp.dot(xe[...], wfb[...], preferred_element_type=jnp.float32)
        y = y + bfb[...]
        mu = jnp.mean(y, axis=0, keepdims=True)
        var = jnp.mean(y * y, axis=0, keepdims=True) - mu * mu
        ybn = (y - mu) * lax.rsqrt(var + 1e-5) * gfb[...] + befb[...]
        zz = jnp.broadcast_to(blb[...], (B, NCLS))
        for c in range(H // LN):
            zc = psum[c, :B, :] / cntc + ybn[:, c * LN:(c + 1) * LN]
            zz = zz + jnp.dot(zc, wlb[c], preferred_element_type=jnp.float32)
        s = jax.nn.sigmoid(zz)
        out[:, 0:NCLS] = s
        out[:, NCLS:2 * NCLS] = s

    z4 = lambda i: (0, 0, 0, 0)
    z3 = lambda i: (0, 0, 0)
    z2 = lambda i: (0, 0)
    return pl.pallas_call(
        body,
        grid=(1,),
        in_specs=[
            pl.BlockSpec((NCORES, H // LN, NPOOL, LN), z4),
            pl.BlockSpec((80, LN), z2),
            pl.BlockSpec((B, SEQ), z2),
            pl.BlockSpec((SEQ, H), z2),
            pl.BlockSpec((1, H), z2),
            pl.BlockSpec((1, H), z2),
            pl.BlockSpec((1, H), z2),
            pl.BlockSpec((H // LN, LN, NCLS), z3),
            pl.BlockSpec((1, NCLS), z2),
        ],
        out_specs=pl.BlockSpec((B, 2 * NCLS), z2),
        out_shape=jax.ShapeDtypeStruct((B, 2 * NCLS), jnp.float32),
    )(poolp, xbat2, x_emb, wf, bf, gf, bef, wl, bl)


def kernel(x_res, x_emb_seq, edge_index, edge_attr, x_batch,
           W1, b1, g1, be1, W2, b2, g2, be2, W3, b3, g3, be3,
           Wf, bf, gf, bef, Wl, bl):
    del edge_attr
    f32 = jnp.float32
    i32 = jnp.int32

    src = edge_index[0]
    dst = edge_index[1]
    epad = NW * NB_E * EB - E
    src3 = jnp.concatenate([src, jnp.zeros((epad,), i32)]).reshape(
        NW, NB_E, EB)
    dst3 = jnp.concatenate([dst, jnp.full((epad,), N, i32)]).reshape(
        NW, NB_E, EB)
    npad_n = NW * NB_P * EB - N
    psrc3 = jnp.concatenate(
        [jnp.arange(N, dtype=i32), jnp.zeros((npad_n,), i32)]).reshape(
            NW, NB_P, EB)
    pdst3 = jnp.concatenate(
        [x_batch, jnp.full((npad_n,), B, i32)]).reshape(NW, NB_P, EB)

    z128 = jnp.asarray(np.zeros((128, LN), np.float32))
    _onc = np.zeros((N, LN), np.float32)
    _onc[:, 0] = 1.0
    ones_nc = jnp.asarray(_onc)

    w1aug = jnp.concatenate(
        [W1.reshape(2, LN, H),
         jnp.concatenate([b1[None, :], jnp.zeros((LN - 1, H), f32)],
                         axis=0)[None]], axis=0)
    w2c = W2.reshape(H // LN, LN, H)
    w3c = W3.reshape(H // LN, LN, H)
    zcol = jnp.zeros((N, 1), f32)

    tok = z128[:8]
    degp = _sc_poolagg(ones_nc[None], src3, dst3, z128, tok, 1,
                       NB_E, NPAD, RPT)
    tok = degp[0, 0, :8]
    x1p, dinv = _tc_prep(x_res, degp)

    agg1p = _sc_poolagg(x1p, src3, dst3, z128, tok, 3, NB_E, NPAD, RPT)
    tok = agg1p[0, 0, :8]
    h1, st1, rowsum = _tc_matmul(agg1p, x1p, dinv, zcol, w1aug,
                                 jnp.zeros((1, H), f32), 3)
    x2p = _tc_finalize(h1, st1, g1.reshape(1, H), be1.reshape(1, H),
                       dinv, True)

    agg2p = _sc_poolagg(x2p, src3, dst3, z128, tok, H // LN, NB_E, NPAD,
                        RPT)
    tok = agg2p[0, 0, :8]
    h2, st2, _ = _tc_matmul(agg2p, x2p, dinv, rowsum, w2c,
                            b2.reshape(1, H), H // LN)
    x3p = _tc_finalize(h2, st2, g2.reshape(1, H), be2.reshape(1, H),
                       dinv, True)

    agg3p = _sc_poolagg(x3p, src3, dst3, z128, tok, H // LN, NB_E, NPAD,
                        RPT)
    tok = agg3p[0, 0, :8]
    h3, st3, _ = _tc_matmul(agg3p, x3p, dinv, rowsum, w3c,
                            b3.reshape(1, H), H // LN)
    x4p = _tc_finalize(h3, st3, g3.reshape(1, H), be3.reshape(1, H),
                       dinv, False)

    poolp = _sc_poolagg(x4p, psrc3, pdst3, z128, tok, H // LN,
                        NB_P, NPOOL, NPOOL // NSUB)
    xbat2 = jnp.concatenate(
        [x_batch, jnp.full((80 * LN - N,), B, i32)]).reshape(80, LN)
    return _tc_head(poolp, xbat2, x_emb_seq,
                    Wf, bf.reshape(1, H), gf.reshape(1, H),
                    bef.reshape(1, H), Wl.reshape(H // LN, LN, NCLS),
                    bl.reshape(1, NCLS))


# speedup vs baseline: 1.2962x; 1.2500x over previous
import functools

import jax
import jax.numpy as jnp
import numpy as np
from jax import lax
from jax.experimental import pallas as pl
from jax.experimental.pallas import tpu as pltpu
from jax.experimental.pallas import tpu_sc as plsc

N = 10000
E = 160000
DIN = 256
H = 1024
SEQ = 256
NCLS = 128
B = 64

NCORES = 2
NSUB = 16
NW = NCORES * NSUB

LN = 128
EB = 128
NB_E = 40
NB_P = 3
NPAD = 10112
RPT = NPAD // NSUB
NPOOL = 128
BN_ROWS = 400
RB = N // BN_ROWS

_mesh = functools.partial(
    plsc.VectorSubcoreMesh,
    core_axis_name="c", subcore_axis_name="s",
    num_cores=NCORES, num_subcores=NSUB)


def _zero_slice(zb, acc, r0, rows):
    done = 0
    while done < rows:
        step = min(40, rows - done)
        pltpu.sync_copy(zb.at[pl.ds(0, step)], acc.at[pl.ds(r0 + done, step)])
        done += step


def _sc_poolagg(xin, src3, dst3, z128, tok, n_chunks, nb, npad_out,
                rpt_out):
    pipelined = nb % 2 == 0
    niter = nb // 2

    def body(x_h, src_h, dst_h, z_h, tok_h, outp, sidx, didx,
             g0, g1, zb, tokv, acc, s0, s1):
        scid = lax.axis_index("c")
        sid = lax.axis_index("s")
        wid = scid * NSUB + sid
        pltpu.sync_copy(src_h.at[wid], sidx)
        pltpu.sync_copy(dst_h.at[wid], didx)
        pltpu.sync_copy(z_h.at[pl.ds(0, 40)], zb)
        pltpu.sync_copy(tok_h, tokv)
        r0 = sid * rpt_out
        for c in range(n_chunks):
            _zero_slice(zb, acc, r0, rpt_out)
            plsc.subcore_barrier()
            if not pipelined:
                def step(i, carry):
                    pltpu.async_copy(
                        x_h.at[c].at[sidx.at[i]], g0, s0).wait()
                    pltpu.sync_copy(g0, acc.at[didx.at[i]], add=True)
                    return carry
                lax.fori_loop(0, nb, step, 0)
            else:
                pltpu.async_copy(x_h.at[c].at[sidx.at[0]], g0, s0)
                pltpu.async_copy(x_h.at[c].at[sidx.at[1]], g1, s1)

                def step(i, carry):
                    for k, (g, s) in enumerate(((g0, s0), (g1, s1))):
                        b = i * 2 + k
                        pltpu.make_async_copy(
                            x_h.at[c].at[sidx.at[b]], g, s).wait()
                        pltpu.sync_copy(g, acc.at[didx.at[b]], add=True)

                        @pl.when(i < niter - 1)
                        def _():
                            pltpu.async_copy(
                                x_h.at[c].at[sidx.at[b + 2]], g, s)
                    return carry
                lax.fori_loop(0, niter, step, 0)
            plsc.subcore_barrier()
            pltpu.sync_copy(acc.at[pl.ds(r0, rpt_out)],
                            outp.at[scid].at[c].at[pl.ds(r0, rpt_out)])

    fn = pl.kernel(
        body,
        out_type=jax.ShapeDtypeStruct(
            (NCORES, n_chunks, npad_out, LN), jnp.float32),
        mesh=_mesh(),
        scratch_types=[
            pltpu.VMEM((nb, EB), jnp.int32),
            pltpu.VMEM((nb, EB), jnp.int32),
            pltpu.VMEM((EB, LN), jnp.float32),
            pltpu.VMEM((EB, LN), jnp.float32),
            pltpu.VMEM((40, LN), jnp.float32),
            pltpu.VMEM((8, LN), jnp.float32),
            pltpu.VMEM_SHARED((npad_out, LN), jnp.float32),
            pltpu.SemaphoreType.DMA,
            pltpu.SemaphoreType.DMA,
        ],
    )
    return fn(xin, src3, dst3, z128, tok)


def _tc_prep(x_res, degp):
    def body(xr, dp, x1p, dinv_o):
        c = pl.program_id(1)
        deg = dp[0, 0, :, 0] + dp[1, 0, :, 0] + 1.0
        dv = lax.rsqrt(jnp.maximum(deg, 1.0))[:, None]
        dinv_o[...] = dv
        cols = lax.broadcasted_iota(jnp.int32, (BN_ROWS, LN), 1)
        dcol = jnp.where(cols == 0, dv, 0.0)
        x1p[0] = jnp.where(c == 2, dcol, xr[...] * dv)

    return pl.pallas_call(
        body,
        grid=(RB, 3),
        in_specs=[
            pl.BlockSpec((BN_ROWS, LN), lambda i, c: (i, jnp.minimum(c, 1))),
            pl.BlockSpec((NCORES, 1, BN_ROWS, LN), lambda i, c: (0, 0, i, 0)),
        ],
        out_specs=[
            pl.BlockSpec((1, BN_ROWS, LN), lambda i, c: (c, i, 0)),
            pl.BlockSpec((BN_ROWS, 1), lambda i, c: (i, 0)),
        ],
        out_shape=[
            jax.ShapeDtypeStruct((3, N, LN), jnp.float32),
            jax.ShapeDtypeStruct((N, 1), jnp.float32),
        ],
    )(x_res, degp)


def _tc_matmul(aggp, xp, dinv, rowsum, wc, bvec, n_chunks):
    def body(ap, xb, dv, rs, w, bv, h_ref, stats_ref, rs_out, stats_acc):
        i = pl.program_id(0)
        c = pl.program_id(1)
        aggc = (ap[0, 0] + ap[1, 0] + xb[0]) * dv[...]
        part = jnp.dot(aggc, w[0], preferred_element_type=jnp.float32)

        @pl.when(c == 0)
        def _():
            h_ref[...] = rs[...] * bv[...] + part

        @pl.when(c != 0)
        def _():
            h_ref[...] += part

        @pl.when(c == n_chunks - 1)
        def _():
            hb = h_ref[...]
            st = jnp.stack([jnp.sum(hb, 0), jnp.sum(hb * hb, 0)])

            @pl.when(i == 0)
            def _():
                stats_acc[...] = st

            @pl.when(i != 0)
            def _():
                stats_acc[...] += st
            rs_out[...] = aggc[:, 0:1]

        stats_ref[...] = stats_acc[...]

    return pl.pallas_call(
        body,
        grid=(RB, n_chunks),
        in_specs=[
            pl.BlockSpec((NCORES, 1, BN_ROWS, LN), lambda i, c: (0, c, i, 0)),
            pl.BlockSpec((1, BN_ROWS, LN), lambda i, c: (c, i, 0)),
            pl.BlockSpec((BN_ROWS, 1), lambda i, c: (i, 0)),
            pl.BlockSpec((BN_ROWS, 1), lambda i, c: (i, 0)),
            pl.BlockSpec((1, LN, H), lambda i, c: (c, 0, 0)),
            pl.BlockSpec((1, H), lambda i, c: (0, 0)),
        ],
        out_specs=[
            pl.BlockSpec((BN_ROWS, H), lambda i, c: (i, 0)),
            pl.BlockSpec((2, H), lambda i, c: (0, 0)),
            pl.BlockSpec((BN_ROWS, 1), lambda i, c: (i, 0)),
        ],
        out_shape=[
            jax.ShapeDtypeStruct((N, H), jnp.float32),
            jax.ShapeDtypeStruct((2, H), jnp.float32),
            jax.ShapeDtypeStruct((N, 1), jnp.float32),
        ],
        scratch_shapes=[pltpu.VMEM((2, H), jnp.float32)],
    )(aggp, xp, dinv, rowsum, wc, bvec)


def _tc_finalize(h, stats, g, be, dinv, scale):
    def body(hb, st, gb, bb, dv, out):
        mu = st[0:1, :] / N
        var = st[1:2, :] / N - mu * mu
        y = (hb[...] - mu) * lax.rsqrt(var + 1e-5) * gb[...] + bb[...]
        y = jnp.maximum(y, 0.0)
        if scale:
            y = y * dv[...]
        out[0] = y

    return pl.pallas_call(
        body,
        grid=(RB, H // LN),
        in_specs=[
            pl.BlockSpec((BN_ROWS, LN), lambda i, c: (i, c)),
            pl.BlockSpec((2, LN), lambda i, c: (0, c)),
            pl.BlockSpec((1, LN), lambda i, c: (0, c)),
            pl.BlockSpec((1, LN), lambda i, c: (0, c)),
            pl.BlockSpec((BN_ROWS, 1), lambda i, c: (i, 0)),
        ],
        out_specs=pl.BlockSpec((1, BN_ROWS, LN), lambda i, c: (c, i, 0)),
        out_shape=jax.ShapeDtypeStruct((H // LN, N, LN), jnp.float32),
    )(h, stats, g, be, dinv)


def _tc_head(poolp, xbat2, x_emb, wf, bf, gf, bef, wl, bl):
    def body(pp, cp, xe, wfb, bfb, gfb, befb, wlb, blb, out):
        psum = pp[0] + pp[1]
        bids = lax.broadcasted_iota(jnp.int32, (B, 80, LN), 0)
        cnt = jnp.sum((cp[...][None, :, :] == bids).astype(jnp.float32),
                      axis=(1, 2))
        cntc = jnp.maximum(cnt, 1.0)[:, None]
        y = jnp.dot(xe[...], wfb[...], preferred_element_type=jnp.float32)
        y = y + bfb[...]
        mu = jnp.mean(y, axis=0, keepdims=True)
        var = jnp.mean(y * y, axis=0, keepdims=True) - mu * mu
        ybn = (y - mu) * lax.rsqrt(var + 1e-5) * gfb[...] + befb[...]
        zz = jnp.broadcast_to(blb[...], (B, NCLS))
        for c in range(H // LN):
            zc = psum[c, :B, :] / cntc + ybn[:, c * LN:(c + 1) * LN]
            zz = zz + jnp.dot(zc, wlb[c], preferred_element_type=jnp.float32)
        s = jax.nn.sigmoid(zz)
        out[:, 0:NCLS] = s
        out[:, NCLS:2 * NCLS] = s

    z4 = lambda i: (0, 0, 0, 0)
    z3 = lambda i: (0, 0, 0)
    z2 = lambda i: (0, 0)
    return pl.pallas_call(
        body,
        grid=(1,),
        in_specs=[
            pl.BlockSpec((NCORES, H // LN, NPOOL, LN), z4),
            pl.BlockSpec((80, LN), z2),
            pl.BlockSpec((B, SEQ), z2),
            pl.BlockSpec((SEQ, H), z2),
            pl.BlockSpec((1, H), z2),
            pl.BlockSpec((1, H), z2),
            pl.BlockSpec((1, H), z2),
            pl.BlockSpec((H // LN, LN, NCLS), z3),
            pl.BlockSpec((1, NCLS), z2),
        ],
        out_specs=pl.BlockSpec((B, 2 * NCLS), z2),
        out_shape=jax.ShapeDtypeStruct((B, 2 * NCLS), jnp.float32),
    )(poolp, xbat2, x_emb, wf, bf, gf, bef, wl, bl)


def kernel(x_res, x_emb_seq, edge_index, edge_attr, x_batch,
           W1, b1, g1, be1, W2, b2, g2, be2, W3, b3, g3, be3,
           Wf, bf, gf, bef, Wl, bl):
    del edge_attr
    f32 = jnp.float32
    i32 = jnp.int32

    src = edge_index[0]
    dst = edge_index[1]
    epad = NW * NB_E * EB - E
    src3 = jnp.concatenate([src, jnp.zeros((epad,), i32)]).reshape(
        NW, NB_E, EB)
    dst3 = jnp.concatenate([dst, jnp.full((epad,), N, i32)]).reshape(
        NW, NB_E, EB)
    npad_n = NW * NB_P * EB - N
    psrc3 = jnp.concatenate(
        [jnp.arange(N, dtype=i32), jnp.zeros((npad_n,), i32)]).reshape(
            NW, NB_P, EB)
    pdst3 = jnp.concatenate(
        [x_batch, jnp.full((npad_n,), B, i32)]).reshape(NW, NB_P, EB)

    z128 = jnp.asarray(np.zeros((128, LN), np.float32))
    _onc = np.zeros((N, LN), np.float32)
    _onc[:, 0] = 1.0
    ones_nc = jnp.asarray(_onc)

    w1aug = jnp.concatenate(
        [W1.reshape(2, LN, H),
         jnp.concatenate([b1[None, :], jnp.zeros((LN - 1, H), f32)],
                         axis=0)[None]], axis=0)
    w2c = W2.reshape(H // LN, LN, H)
    w3c = W3.reshape(H // LN, LN, H)
    zcol = jnp.zeros((N, 1), f32)

    tok = z128[:8]
    degp = _sc_poolagg(ones_nc[None], src3, dst3, z128, tok, 1,
                       NB_E, NPAD, RPT)
    tok = degp[0, 0, :8]
    x1p, dinv = _tc_prep(x_res, degp)

    agg1p = _sc_poolagg(x1p, src3, dst3, z128, tok, 3, NB_E, NPAD, RPT)
    tok = agg1p[0, 0, :8]
    h1, st1, rowsum = _tc_matmul(agg1p, x1p, dinv, zcol, w1aug,
                                 jnp.zeros((1, H), f32), 3)
    x2p = _tc_finalize(h1, st1, g1.reshape(1, H), be1.reshape(1, H),
                       dinv, True)

    agg2p = _sc_poolagg(x2p, src3, dst3, z128, tok, H // LN, NB_E, NPAD,
                        RPT)
    tok = agg2p[0, 0, :8]
    h2, st2, _ = _tc_matmul(agg2p, x2p, dinv, rowsum, w2c,
                            b2.reshape(1, H), H // LN)
    x3p = _tc_finalize(h2, st2, g2.reshape(1, H), be2.reshape(1, H),
                       dinv, True)

    agg3p = _sc_poolagg(x3p, src3, dst3, z128, tok, H // LN, NB_E, NPAD,
                        RPT)
    tok = agg3p[0, 0, :8]
    h3, st3, _ = _tc_matmul(agg3p, x3p, dinv, rowsum, w3c,
                            b3.reshape(1, H), H // LN)
    x4p = _tc_finalize(h3, st3, g3.reshape(1, H), be3.reshape(1, H),
                       dinv, False)

    poolp = _sc_poolagg(x4p, psrc3, pdst3, z128, tok, H // LN,
                        NB_P, NPOOL, NPOOL // NSUB)
    xbat2 = jnp.concatenate(
        [x_batch, jnp.full((80 * LN - N,), B, i32)]).reshape(80, LN)
    return _tc_head(poolp, xbat2, x_emb_seq,
                    Wf, bf.reshape(1, H), gf.reshape(1, H),
                    bef.reshape(1, H), Wl.reshape(H // LN, LN, NCLS),
                    bl.reshape(1, NCLS))
